# baseline (speedup 1.0000x reference)
import jax
import jax.numpy as jnp
from jax.experimental import pallas as pl
from jax.experimental.pallas import tpu as pltpu

_NEG = -1e9
_B, _S, _T, _V = 16, 24, 32, 2
_L = _S * _B


def _dp_kernel(rst0, rst1, lst0, lst1, bd, u, af,
               c0d0t, c0d1e, c1d0e, c1d1t, hdt, hde, er0, er1, el0, el1,
               exq0, eyq0, exq1, eyq1, s00a, s01a, s10a, s11a):
    s, t, L = _S, _T, _L
    f32 = jnp.float32
    roll = lambda x, k: pltpu.roll(x, k % L, axis=x.ndim - 1)

    mrx = jnp.max(jnp.maximum(rst0[...], rst1[...]), axis=0)
    er0[...] = jnp.exp(rst0[...] - mrx[None])
    er1[...] = jnp.exp(rst1[...] - mrx[None])
    mlx = jnp.max(jnp.maximum(lst0[...], lst1[...]), axis=0)
    el0[...] = jnp.exp(lst0[...] - mlx[None])
    el1[...] = jnp.exp(lst1[...] - mlx[None])

    ub = u[...]
    c0d0t[0] = bd[0, 0, 0] + ub
    c0d1e[0] = bd[0, 1, 0] + ub
    c1d0e[0] = bd[1, 0, 0] + ub
    c1d1t[0] = bd[1, 1, 0] + ub
    h0 = c0d0t[0] + c1d0e[0]
    hdt[0] = h0
    hde[0] = h0
    af[0] = (bd[1, 0, 0] + ub)[0, 0:_B]

    for ln in range(1, s):
        r = ln * _B

        def pre_body(q, carry):
            mq0, mq1 = carry
            x0 = hdt[q]
            a0 = jnp.max(x0, axis=0, keepdims=True)
            exq0[q] = jnp.exp(x0 - a0)
            y0 = roll(c0d1e[ln - 1 - q], -r)
            b0 = jnp.max(y0, axis=0, keepdims=True)
            eyq0[q] = y0 + a0
            x1 = c1d1t[q]
            a1 = jnp.max(x1, axis=0, keepdims=True)
            exq1[q] = jnp.exp(x1 - a1)
            y1 = roll(hde[ln - 1 - q], -r)
            b1 = jnp.max(y1, axis=0, keepdims=True)
            eyq1[q] = y1 + a1
            return jnp.maximum(mq0, a0 + b0), jnp.maximum(mq1, a1 + b1)

        init = (jnp.full((1, L), _NEG, f32), jnp.full((1, L), _NEG, f32))
        mq0, mq1 = jax.lax.fori_loop(0, ln, pre_body, init)

        def expb_body(q, _):
            eyq0[q] = jnp.exp(eyq0[q] - mq0)
            eyq1[q] = jnp.exp(eyq1[q] - mq1)
            return 0

        jax.lax.fori_loop(0, ln, expb_body, 0)

        s00a[...] = jnp.zeros((t, L), f32)
        s01a[...] = jnp.zeros((t, L), f32)
        s10a[...] = jnp.zeros((t, L), f32)
        s11a[...] = jnp.zeros((t, L), f32)
        nb = 2
        for blk in range(0, t, nb):
            w = min(nb, t - blk)

            def blk0_body(q, acc0):
                rows0 = exq0[q, blk:blk + w]
                return acc0 + jnp.broadcast_to(rows0[:, None], (w, t, L)) * eyq0[q][None]

            acc0 = jax.lax.fori_loop(0, ln, blk0_body, jnp.zeros((w, t, L), f32))
            acc0 = roll(acc0, r)
            s00a[...] += jnp.sum(er0[blk:blk + w] * acc0, axis=0)
            s01a[...] += jnp.sum(er1[blk:blk + w] * acc0, axis=0)

            def blk1_body(q, acc1):
                rows1 = eyq1[q, blk:blk + w]
                return acc1 + jnp.broadcast_to(rows1[:, None], (w, t, L)) * exq1[q][None]

            acc1 = jax.lax.fori_loop(0, ln, blk1_body, jnp.zeros((w, t, L), f32))
            s10a[...] += jnp.sum(el0[blk:blk + w] * acc1, axis=0)
            s11a[...] += jnp.sum(el1[blk:blk + w] * acc1, axis=0)

        sh0 = roll(mq0, r) + mrx
        out00 = bd[0, 0, 1] + sh0 + jnp.log(s00a[...])
        out01 = bd[0, 1, 1] + sh0 + jnp.log(s01a[...])
        c0d0t[ln] = roll(out00, -r)
        c0d1e[ln] = out01
        sh1 = mq1 + mlx
        s10 = s10a[...]
        s11 = s11a[...]
        out10 = bd[1, 0, 1] + sh1 + jnp.log(s10)
        out11 = bd[1, 1, 1] + sh1 + jnp.log(s11)
        c1d0e[ln] = roll(out10, r)
        c1d1t[ln] = out11
        af[ln] = out10[0, 0:_B]

        def hm_body(k, mz):
            return jnp.maximum(mz, c0d0t[k] + roll(c1d0e[ln - k], -r))

        mz = jax.lax.fori_loop(0, ln + 1, hm_body, jnp.full((t, L), _NEG, f32))

        def hacc_body(k, acc):
            return acc + jnp.exp(c0d0t[k] + roll(c1d0e[ln - k], -r) - mz)

        sha = jax.lax.fori_loop(0, ln + 1, hacc_body, jnp.zeros((t, L), f32))
        hrow = mz + jnp.log(sha)
        hdt[ln] = hrow
        hde[ln] = roll(hrow, r)


def _run_dp(rst0, rst1, lst0, lst1, bd, u):
    s, t, L = _S, _T, _L
    f32 = jnp.float32
    scratch = [
        pltpu.VMEM((s, t, L), f32),
        pltpu.VMEM((s, t, L), f32),
        pltpu.VMEM((s, t, L), f32),
        pltpu.VMEM((s, t, L), f32),
        pltpu.VMEM((s, t, L), f32),
        pltpu.VMEM((s, t, L), f32),
        pltpu.VMEM((t, t, L), f32),
        pltpu.VMEM((t, t, L), f32),
        pltpu.VMEM((t, t, L), f32),
        pltpu.VMEM((t, t, L), f32),
        pltpu.VMEM((s, t, L), f32),
        pltpu.VMEM((s, t, L), f32),
        pltpu.VMEM((s, t, L), f32),
        pltpu.VMEM((s, t, L), f32),
        pltpu.VMEM((t, L), f32),
        pltpu.VMEM((t, L), f32),
        pltpu.VMEM((t, L), f32),
        pltpu.VMEM((t, L), f32),
    ]
    return pl.pallas_call(
        _dp_kernel,
        out_shape=jax.ShapeDtypeStruct((s, _B), f32),
        scratch_shapes=scratch,
    )(rst0, rst1, lst0, lst1, bd, u)


def kernel(left_score, right_score, batch_decision_score, batch_unary_score, sent_lens):
    b, s, t, _ = left_score.shape
    ls5 = left_score.reshape(b, s, t, t, _V)
    rs5 = right_score.reshape(b, s, t, t, _V)
    rst0 = rs5[..., 0].transpose(2, 3, 1, 0).reshape(t, t, s * b)
    rst1 = rs5[..., 1].transpose(2, 3, 1, 0).reshape(t, t, s * b)
    lst0 = ls5[..., 0].transpose(3, 2, 1, 0).reshape(t, t, s * b)
    lst1 = ls5[..., 1].transpose(3, 2, 1, 0).reshape(t, t, s * b)
    bd = batch_decision_score.transpose(3, 4, 5, 2, 1, 0).reshape(2, _V, 2, t, s * b)
    u = batch_unary_score.transpose(2, 1, 0).reshape(t, s * b)
    af = _run_dp(rst0, rst1, lst0, lst1, bd, u)
    return af[sent_lens - 1, jnp.arange(b)]


# speedup vs baseline: 1.0414x; 1.0414x over previous
import jax
import jax.numpy as jnp
from jax.experimental import pallas as pl
from jax.experimental.pallas import tpu as pltpu

_NEG = -1e9
_B, _S, _T, _V = 16, 24, 32, 2
_L = _S * _B


def _dp_kernel(rst0, rst1, lst0, lst1, bd, u, af,
               c0d0t, c0d1e, c1d0e, c1d1t, hdt, hde, er0, er1, el0, el1, g0, g1):
    s, t, L = _S, _T, _L
    f32 = jnp.float32
    roll = lambda x, k: pltpu.roll(x, k % L, axis=x.ndim - 1)

    mrx = jnp.max(jnp.maximum(rst0[...], rst1[...]), axis=0)
    er0[...] = jnp.exp(rst0[...] - mrx[None])
    er1[...] = jnp.exp(rst1[...] - mrx[None])
    mlx = jnp.max(jnp.maximum(lst0[...], lst1[...]), axis=0)
    el0[...] = jnp.exp(lst0[...] - mlx[None])
    el1[...] = jnp.exp(lst1[...] - mlx[None])

    ub = u[...]
    c0d0t[0] = bd[0, 0, 0] + ub
    c0d1e[0] = bd[0, 1, 0] + ub
    c1d0e[0] = bd[1, 0, 0] + ub
    c1d1t[0] = bd[1, 1, 0] + ub
    h0 = c0d0t[0] + c1d0e[0]
    hdt[0] = h0
    hde[0] = h0
    af[0] = (bd[1, 0, 0] + ub)[0, 0:_B]

    for ln in range(1, s):
        r = ln * _B

        def mx_body(q, carry):
            mq0, mq1 = carry
            a0 = jnp.max(hdt[q], axis=0, keepdims=True)
            b0 = jnp.max(roll(c0d1e[ln - 1 - q], -r), axis=0, keepdims=True)
            a1 = jnp.max(c1d1t[q], axis=0, keepdims=True)
            b1 = jnp.max(roll(hde[ln - 1 - q], -r), axis=0, keepdims=True)
            return jnp.maximum(mq0, a0 + b0), jnp.maximum(mq1, a1 + b1)

        init = (jnp.full((1, L), _NEG, f32), jnp.full((1, L), _NEG, f32))
        mq0, mq1 = jax.lax.fori_loop(0, ln, mx_body, init)

        g0[...] = jnp.zeros((t, t, L), f32)
        g1[...] = jnp.zeros((t, t, L), f32)

        def acc_body(q, _):
            x0 = hdt[q]
            a0 = jnp.max(x0, axis=0, keepdims=True)
            ex0 = jnp.exp(x0 - a0)
            y0 = roll(c0d1e[ln - 1 - q], -r)
            ey0 = jnp.exp(y0 + a0 - mq0)
            g0[...] += jnp.broadcast_to(ex0[:, None], (t, t, L)) * ey0[None]
            x1 = c1d1t[q]
            a1 = jnp.max(x1, axis=0, keepdims=True)
            ex1 = jnp.exp(x1 - a1)
            y1 = roll(hde[ln - 1 - q], -r)
            ey1 = jnp.exp(y1 + a1 - mq1)
            g1[...] += jnp.broadcast_to(ey1[:, None], (t, t, L)) * ex1[None]
            return 0

        jax.lax.fori_loop(0, ln, acc_body, 0)

        g0e = roll(g0[...], r)
        sh0 = roll(mq0, r) + mrx
        s00 = jnp.sum(er0[...] * g0e, axis=0)
        s01 = jnp.sum(er1[...] * g0e, axis=0)
        out00 = bd[0, 0, 1] + sh0 + jnp.log(s00)
        out01 = bd[0, 1, 1] + sh0 + jnp.log(s01)
        c0d0t[ln] = roll(out00, -r)
        c0d1e[ln] = out01
        G1 = g1[...]
        sh1 = mq1 + mlx
        s10 = jnp.sum(el0[...] * G1, axis=0)
        s11 = jnp.sum(el1[...] * G1, axis=0)
        out10 = bd[1, 0, 1] + sh1 + jnp.log(s10)
        out11 = bd[1, 1, 1] + sh1 + jnp.log(s11)
        c1d0e[ln] = roll(out10, r)
        c1d1t[ln] = out11
        af[ln] = out10[0, 0:_B]

        def hm_body(k, mz):
            return jnp.maximum(mz, c0d0t[k] + roll(c1d0e[ln - k], -r))

        mz = jax.lax.fori_loop(0, ln + 1, hm_body, jnp.full((t, L), _NEG, f32))

        def hacc_body(k, acc):
            return acc + jnp.exp(c0d0t[k] + roll(c1d0e[ln - k], -r) - mz)

        sha = jax.lax.fori_loop(0, ln + 1, hacc_body, jnp.zeros((t, L), f32))
        hrow = mz + jnp.log(sha)
        hdt[ln] = hrow
        hde[ln] = roll(hrow, r)


def _run_dp(rst0, rst1, lst0, lst1, bd, u):
    s, t, L = _S, _T, _L
    f32 = jnp.float32
    scratch = [
        pltpu.VMEM((s, t, L), f32),
        pltpu.VMEM((s, t, L), f32),
        pltpu.VMEM((s, t, L), f32),
        pltpu.VMEM((s, t, L), f32),
        pltpu.VMEM((s, t, L), f32),
        pltpu.VMEM((s, t, L), f32),
        pltpu.VMEM((t, t, L), f32),
        pltpu.VMEM((t, t, L), f32),
        pltpu.VMEM((t, t, L), f32),
        pltpu.VMEM((t, t, L), f32),
        pltpu.VMEM((t, t, L), f32),
        pltpu.VMEM((t, t, L), f32),
    ]
    return pl.pallas_call(
        _dp_kernel,
        out_shape=jax.ShapeDtypeStruct((s, _B), f32),
        scratch_shapes=scratch,
    )(rst0, rst1, lst0, lst1, bd, u)


def kernel(left_score, right_score, batch_decision_score, batch_unary_score, sent_lens):
    b, s, t, _ = left_score.shape
    ls5 = left_score.reshape(b, s, t, t, _V)
    rs5 = right_score.reshape(b, s, t, t, _V)
    rst0 = rs5[..., 0].transpose(2, 3, 1, 0).reshape(t, t, s * b)
    rst1 = rs5[..., 1].transpose(2, 3, 1, 0).reshape(t, t, s * b)
    lst0 = ls5[..., 0].transpose(3, 2, 1, 0).reshape(t, t, s * b)
    lst1 = ls5[..., 1].transpose(3, 2, 1, 0).reshape(t, t, s * b)
    bd = batch_decision_score.transpose(3, 4, 5, 2, 1, 0).reshape(2, _V, 2, t, s * b)
    u = batch_unary_score.transpose(2, 1, 0).reshape(t, s * b)
    af = _run_dp(rst0, rst1, lst0, lst1, bd, u)
    return af[sent_lens - 1, jnp.arange(b)]
